# baseline (speedup 1.0000x reference)
import functools

import jax
import jax.numpy as jnp
from jax import lax
from jax.experimental import pallas as pl
from jax.experimental.pallas import tpu as pltpu
from jax.experimental.pallas import tpu_sc as plsc

_N = 50000
_E = 800000
_NPAD = 51200
_EPT = _E // 16
_CH = 2000
_NCHUNK = _EPT // _CH
_ZR = 16
_EPS = 1e-9
_BLK = 2048


def _make_sc_gat(row, fe, nheads, fph, ws, nwin_per_sc, kk):
    mesh = plsc.VectorSubcoreMesh(
        core_axis_name="c", subcore_axis_name="s", num_cores=2, num_subcores=16
    )
    rpt = ws // 16
    cap = _CH + kk + 48
    nsan = kk // 16 + 1

    @functools.partial(
        pl.kernel,
        out_type=jax.ShapeDtypeStruct((_NPAD, row), jnp.float32),
        mesh=mesh,
        compiler_params=pltpu.CompilerParams(needs_layout_passes=False,
                                             use_tc_tiling_on_sc=False),
        scratch_types=[
            pltpu.VMEM((ws * nheads,), jnp.float32),
            pltpu.VMEM((_CH,), jnp.int32),
            pltpu.VMEM((_CH,), jnp.int32),
            pltpu.VMEM((cap,), jnp.int32),
            pltpu.VMEM((cap,), jnp.int32),
            pltpu.VMEM((3, kk), jnp.int32),
            pltpu.VMEM((kk, row), jnp.float32),
            pltpu.VMEM((kk, row), jnp.float32),
            pltpu.VMEM((kk, row), jnp.float32),
            pltpu.VMEM_SHARED((ws, row), jnp.float32),
            pltpu.SemaphoreType.DMA,
            pltpu.SemaphoreType.DMA,
            pltpu.SemaphoreType.DMA,
            pltpu.SemaphoreType.DMA,
            pltpu.SemaphoreType.DMA,
            pltpu.SemaphoreType.DMA,
        ],
    )
    def sc_gat(fe_hbm, er_hbm, src_hbm, dst_hbm, out_hbm,
               er_win, srcv, dstv, st_src, st_dst, didx, rb0, rb1, rb2,
               acc, semg0, semg1, semg2, sems0, sems1, sems2):
        cid = lax.axis_index("c")
        sid = lax.axis_index("s")
        iota = lax.iota(jnp.int32, 16)
        zf = jnp.zeros((16,), jnp.float32)
        zi = jnp.zeros((16,), jnp.int32)
        bufs = (rb0, rb1, rb2)
        sems = (semg0, semg1, semg2)
        ssems = (sems0, sems1, sems2)

        def fire(b, phase):
            pltpu.async_copy(
                fe_hbm.at[st_src.at[pl.ds(b * kk, kk)]], bufs[phase],
                sems[phase])

        def drain(b, phase):
            pltpu.make_async_copy(
                fe_hbm.at[st_src.at[pl.ds(b * kk, kk)]], bufs[phase],
                sems[phase]).wait()

        def fire_scat(phase):
            pltpu.async_copy(bufs[phase], acc.at[didx.at[phase]],
                             ssems[phase], add=True)

        def drain_scat(phase):
            pltpu.make_async_copy(bufs[phase], acc.at[didx.at[phase]],
                                  ssems[phase]).wait()

        def process(b, phase, k):
            buf = bufs[phase]

            @plsc.parallel_loop(0, kk // 16)
            def _grp(g):
                base = b * kk + g * 16
                rows16 = g * 16 + iota
                dl16 = st_dst[pl.ds(base, 16)]
                didx[phase, pl.ds(g * 16, 16)] = dl16
                valid = (base + iota) < k
                wv_h = []
                for hh in range(nheads):
                    col = jnp.full((16,), fe + hh, jnp.int32)
                    el = plsc.load_gather(buf, [rows16, col])
                    er = plsc.load_gather(er_win, [dl16 * nheads + hh])
                    z = el + er
                    wv = jnp.exp(jnp.maximum(z, 0.2 * z))
                    wv = jnp.where(valid, wv, 0.0)
                    plsc.store_scatter(buf, [rows16, col], wv)
                    wv_h.append(wv)
                for rl in range(16):
                    r = g * 16 + rl
                    rlv = jnp.full((16,), rl, jnp.int32)
                    for hh in range(nheads):
                        wb = wv_h[hh].at[rlv].get(mode="promise_in_bounds")
                        for cg in range(fph // 16):
                            c0 = hh * fph + cg * 16
                            buf[r, pl.ds(c0, 16)] = buf[r, pl.ds(c0, 16)] * wb
            fire_scat(phase)

        def window_body(wloc, _):
            lo = (cid * nwin_per_sc + wloc) * ws

            for r in range(_ZR):
                for cg in range(row // 16):
                    rb0[r, pl.ds(cg * 16, 16)] = zf

            def zero_body(i, _, base=sid * rpt):
                pltpu.sync_copy(rb0.at[pl.ds(0, _ZR)],
                                acc.at[pl.ds(base + i * _ZR, _ZR)])
                return 0
            lax.fori_loop(0, rpt // _ZR, zero_body, 0)
            pltpu.sync_copy(er_hbm.at[pl.ds(lo * nheads, ws * nheads)], er_win)
            plsc.subcore_barrier()

            def chunk_body(c, _, lo=lo):
                off = sid * _EPT + c * _CH
                pltpu.sync_copy(src_hbm.at[pl.ds(off, _CH)], srcv)
                pltpu.sync_copy(dst_hbm.at[pl.ds(off, _CH)], dstv)

                lane15 = jnp.full((16,), 15, jnp.int32)

                def comp_body(g, kvec):
                    d16 = dstv[pl.ds(g * 16, 16)]
                    s16 = srcv[pl.ds(g * 16, 16)]
                    m = (d16 >= lo) & (d16 < lo + ws)
                    p = jnp.where(m, 1, 0)
                    for s in (1, 2, 4, 8):
                        sh = p.at[jnp.maximum(iota - s, 0)].get(
                            mode="promise_in_bounds")
                        p = p + jnp.where(iota >= s, sh, 0)
                    trash = cap - 16 + iota
                    idx = jnp.where(m, kvec + p - 1, trash)
                    plsc.store_scatter(st_src, [idx], s16)
                    plsc.store_scatter(st_dst, [idx], d16 - lo)
                    return kvec + p.at[lane15].get(mode="promise_in_bounds")
                kvec = lax.fori_loop(0, _CH // 16, comp_body,
                                     jnp.zeros((16,), jnp.int32))
                k = kvec[0]

                for j in range(nsan):
                    sidx = k + j * 16 + iota
                    plsc.store_scatter(st_src, [sidx], zi)
                    plsc.store_scatter(st_dst, [sidx], zi)

                nb = (k + kk - 1) // kk

                for p in range(3):
                    @pl.when(p < nb)
                    def _(p=p):
                        fire(p, p)

                def tri_body(t, _, k=k):
                    for p in range(3):
                        b = 3 * t + p

                        @pl.when(b < nb)
                        def _(b=b, p=p):
                            drain(b, p)
                            process(b, p, k)

                            @pl.when(b + 3 < nb)
                            def _():
                                drain_scat(p)
                                fire(b + 3, p)
                    return 0
                lax.fori_loop(0, (nb + 2) // 3, tri_body, 0)
                for p in range(3):
                    @pl.when(p < nb)
                    def _(p=p):
                        drain_scat(p)
                return 0
            lax.fori_loop(0, _NCHUNK, chunk_body, 0)
            plsc.subcore_barrier()

            def flush_body(i, _, base=sid * rpt, lo=lo):
                o = base + i * _ZR
                pltpu.sync_copy(acc.at[pl.ds(o, _ZR)],
                                out_hbm.at[pl.ds(lo + o, _ZR)])
                return 0
            lax.fori_loop(0, rpt // _ZR, flush_body, 0)
            return 0
        lax.fori_loop(0, nwin_per_sc, window_body, 0)

    return sc_gat


_sc_gat1 = _make_sc_gat(row=144, fe=128, nheads=2, fph=64, ws=6400,
                        nwin_per_sc=4, kk=32)
_sc_gat2 = _make_sc_gat(row=48, fe=32, nheads=1, fph=32, ws=25600,
                        nwin_per_sc=1, kk=32)


def _t1_body(x_ref, w1t_ref, al_ref, ar_ref, fe_ref, er_ref):
    feat = jnp.dot(x_ref[...], w1t_ref[...], preferred_element_type=jnp.float32)
    el = jnp.dot(feat, al_ref[...], preferred_element_type=jnp.float32)
    er = jnp.dot(feat, ar_ref[...], preferred_element_type=jnp.float32)
    fe_ref[...] = jnp.concatenate(
        [feat, el, jnp.zeros((feat.shape[0], 14), jnp.float32)], axis=1)
    er_ref[...] = er


def _t2_body(acc_ref, x_ref, b1_ref, s1_ref, t1_ref, w2t_ref, al2_ref, ar2_ref,
             fe2_ref, er2_ref, h1_ref):
    a = acc_ref[...]
    d0 = jnp.maximum(a[:, 128:129], _EPS)
    d1 = jnp.maximum(a[:, 129:130], _EPS)
    rst = jnp.concatenate([a[:, 0:64] / d0, a[:, 64:128] / d1], axis=1)
    rst = rst + x_ref[...] + b1_ref[...]
    h1 = jnp.maximum(rst * s1_ref[...] + t1_ref[...], 0.0)
    feat2 = jnp.dot(h1, w2t_ref[...], preferred_element_type=jnp.float32)
    el2 = jnp.dot(feat2, al2_ref[...], preferred_element_type=jnp.float32)
    er2 = jnp.dot(feat2, ar2_ref[...], preferred_element_type=jnp.float32)
    fe2_ref[...] = jnp.concatenate(
        [feat2, el2, jnp.zeros((feat2.shape[0], 15), jnp.float32)], axis=1)
    er2_ref[...] = er2
    h1_ref[...] = h1


def kernel(x, edge_index, params):
    f32 = jnp.float32
    n = x.shape[0]
    bn_s = 1.0 / jnp.sqrt(1.0 + 1e-5)

    w1t = params['W1'].T
    al1 = params['attn_l1']
    ar1 = params['attn_r1']
    z64 = jnp.zeros((64, 1), f32)
    al1m = jnp.block([[al1[0].reshape(64, 1), z64], [z64, al1[1].reshape(64, 1)]])
    ar1m = jnp.block([[ar1[0].reshape(64, 1), z64], [z64, ar1[1].reshape(64, 1)]])
    b1 = params['bias1'].reshape(1, 128)
    s1 = (params['bn1_g'] * bn_s).reshape(1, 128)
    t1 = params['bn1_b'].reshape(1, 128)
    w2t = params['W2'].T
    al2m = params['attn_l2'].reshape(32, 1)
    ar2m = params['attn_r2'].reshape(32, 1)
    rw2t = params['resW2'].T
    b2 = params['bias2'].reshape(1, 32)
    s2 = (params['bn2_g'] * bn_s).reshape(1, 32)
    t2 = params['bn2_b'].reshape(1, 32)
    dw1t = params['dec_W1'].T
    db1 = params['dec_b1'].reshape(1, 128)
    sd = (params['dec_bn_g'] * bn_s).reshape(1, 128)
    td = params['dec_bn_b'].reshape(1, 128)
    dw2t = params['dec_W2'].T
    db2 = params['dec_b2'].reshape(1, 128)

    xp = jnp.pad(x, ((0, _NPAD - n), (0, 0)))

    grid = (_NPAD // _BLK,)
    full = lambda r, c: pl.BlockSpec((r, c), lambda i: (0, 0))
    rows = lambda c: pl.BlockSpec((_BLK, c), lambda i: (i, 0))

    fe1, er1 = pl.pallas_call(
        _t1_body,
        grid=grid,
        in_specs=[rows(128), full(128, 128), full(128, 2), full(128, 2)],
        out_specs=[rows(144), rows(2)],
        out_shape=[jax.ShapeDtypeStruct((_NPAD, 144), f32),
                   jax.ShapeDtypeStruct((_NPAD, 2), f32)],
    )(xp, w1t, al1m, ar1m)

    src = edge_index[0]
    dst = edge_index[1]
    acc1 = _sc_gat1(fe1, er1.reshape(-1), src, dst)

    fe2, er2, h1 = pl.pallas_call(
        _t2_body,
        grid=grid,
        in_specs=[rows(144), rows(128), full(1, 128), full(1, 128),
                  full(1, 128), full(128, 32), full(32, 1), full(32, 1)],
        out_specs=[rows(48), rows(1), rows(128)],
        out_shape=[jax.ShapeDtypeStruct((_NPAD, 48), f32),
                   jax.ShapeDtypeStruct((_NPAD, 1), f32),
                   jax.ShapeDtypeStruct((_NPAD, 128), f32)],
    )(acc1, xp, b1, s1, t1, w2t, al2m, ar2m)

    acc2 = _sc_gat2(fe2, er2.reshape(-1), src, dst)

    gacc, rec = pl.pallas_call(
        _t3_full_body,
        grid=grid,
        in_specs=[rows(48), rows(128), full(128, 32), full(1, 32), full(1, 32),
                  full(1, 32), full(32, 128), full(1, 128), full(1, 128),
                  full(1, 128), full(128, 128), full(1, 128)],
        out_specs=[pl.BlockSpec((8, 32), lambda i: (0, 0)), rows(128)],
        out_shape=[jax.ShapeDtypeStruct((8, 32), f32),
                   jax.ShapeDtypeStruct((_NPAD, 128), f32)],
    )(acc2, h1, rw2t, b2, s2, t2, dw1t, db1, sd, td, dw2t, db2)

    graph_rep = gacc[0:1] / float(_N)
    return (graph_rep, rec[:n])


def _t3_full_body(acc2_ref, h1_ref, rw2t_ref, b2_ref, s2_ref, t2_ref,
                  dw1t_ref, db1_ref, sd_ref, td_ref, dw2t_ref, db2_ref,
                  g_ref, rec_ref):
    i = pl.program_id(0)
    a = acc2_ref[...]
    den = jnp.maximum(a[:, 32:33], _EPS)
    rst = a[:, 0:32] / den
    rst = rst + jnp.dot(h1_ref[...], rw2t_ref[...],
                        preferred_element_type=jnp.float32) + b2_ref[...]
    h2 = jnp.maximum(rst * s2_ref[...] + t2_ref[...], 0.0)
    row = i * _BLK + lax.broadcasted_iota(jnp.int32, (_BLK, 1), 0)
    part = jnp.sum(jnp.where(row < _N, h2, 0.0), axis=0, keepdims=True)
    part8 = jnp.concatenate([part, jnp.zeros((7, 32), jnp.float32)], axis=0)

    @pl.when(i == 0)
    def _():
        g_ref[...] = jnp.zeros_like(g_ref)

    g_ref[...] += part8
    d = jnp.dot(h2, dw1t_ref[...], preferred_element_type=jnp.float32)
    d = d + db1_ref[...]
    d = jnp.maximum(d * sd_ref[...] + td_ref[...], 0.0)
    rec_ref[...] = jnp.dot(d, dw2t_ref[...],
                           preferred_element_type=jnp.float32) + db2_ref[...]


# speedup vs baseline: 1.4483x; 1.4483x over previous
import functools

import jax
import jax.numpy as jnp
from jax import lax
from jax.experimental import pallas as pl
from jax.experimental.pallas import tpu as pltpu
from jax.experimental.pallas import tpu_sc as plsc

_N = 50000
_E = 800000
_NPAD = 51200
_EPT = _E // 16
_CH = 2000
_NCHUNK = _EPT // _CH
_ZR = 16
_EPS = 1e-9
_BLK = 2048


def _make_sc_gat(row, fe, nheads, fph, ws, nwin_per_sc, kk):
    mesh = plsc.VectorSubcoreMesh(
        core_axis_name="c", subcore_axis_name="s", num_cores=2, num_subcores=16
    )
    rpt = ws // 16
    cap = _CH + kk + 48
    nsan = kk // 16 + 1

    @functools.partial(
        pl.kernel,
        out_type=jax.ShapeDtypeStruct((_NPAD, row), jnp.float32),
        mesh=mesh,
        compiler_params=pltpu.CompilerParams(needs_layout_passes=False,
                                             use_tc_tiling_on_sc=False),
        scratch_types=[
            pltpu.VMEM((ws * nheads,), jnp.float32),
            pltpu.VMEM((_CH,), jnp.int32),
            pltpu.VMEM((_CH,), jnp.int32),
            pltpu.VMEM((cap,), jnp.int32),
            pltpu.VMEM((cap,), jnp.int32),
            pltpu.VMEM((3, kk), jnp.int32),
            pltpu.VMEM((kk, row), jnp.float32),
            pltpu.VMEM((kk, row), jnp.float32),
            pltpu.VMEM((kk, row), jnp.float32),
            pltpu.VMEM_SHARED((ws, row), jnp.float32),
            pltpu.SemaphoreType.DMA,
            pltpu.SemaphoreType.DMA,
            pltpu.SemaphoreType.DMA,
            pltpu.SemaphoreType.DMA,
            pltpu.SemaphoreType.DMA,
            pltpu.SemaphoreType.DMA,
        ],
    )
    def sc_gat(fe_hbm, er_hbm, src_hbm, dst_hbm, out_hbm,
               er_win, srcv, dstv, st_src, st_dst, didx, rb0, rb1, rb2,
               acc, semg0, semg1, semg2, sems0, sems1, sems2):
        cid = lax.axis_index("c")
        sid = lax.axis_index("s")
        iota = lax.iota(jnp.int32, 16)
        zf = jnp.zeros((16,), jnp.float32)
        zi = jnp.zeros((16,), jnp.int32)
        bufs = (rb0, rb1, rb2)
        sems = (semg0, semg1, semg2)
        ssems = (sems0, sems1, sems2)

        def fire(b, phase):
            pltpu.async_copy(
                fe_hbm.at[st_src.at[pl.ds(b * kk, kk)]], bufs[phase],
                sems[phase])

        def drain(b, phase):
            pltpu.make_async_copy(
                fe_hbm.at[st_src.at[pl.ds(b * kk, kk)]], bufs[phase],
                sems[phase]).wait()

        def fire_scat(phase):
            pltpu.async_copy(bufs[phase], acc.at[didx.at[phase]],
                             ssems[phase], add=True)

        def drain_scat(phase):
            pltpu.make_async_copy(bufs[phase], acc.at[didx.at[phase]],
                                  ssems[phase]).wait()

        def process(b, phase, k):
            buf = bufs[phase]

            @plsc.parallel_loop(0, kk // 16)
            def _grp(g):
                base = b * kk + g * 16
                rows16 = g * 16 + iota
                dl16 = st_dst[pl.ds(base, 16)]
                didx[phase, pl.ds(g * 16, 16)] = dl16
                valid = (base + iota) < k
                wv_h = []
                for hh in range(nheads):
                    col = jnp.full((16,), fe + hh, jnp.int32)
                    el = plsc.load_gather(buf, [rows16, col])
                    er = plsc.load_gather(er_win, [dl16 * nheads + hh])
                    z = el + er
                    wv = jnp.exp(jnp.maximum(z, 0.2 * z))
                    wv = jnp.where(valid, wv, 0.0)
                    plsc.store_scatter(buf, [rows16, col], wv)
                    wv_h.append(wv)
                for rl in range(16):
                    r = g * 16 + rl
                    rlv = jnp.full((16,), rl, jnp.int32)
                    for hh in range(nheads):
                        wb = wv_h[hh].at[rlv].get(mode="promise_in_bounds")
                        for cg in range(fph // 16):
                            c0 = hh * fph + cg * 16
                            buf[r, pl.ds(c0, 16)] = buf[r, pl.ds(c0, 16)] * wb
            fire_scat(phase)

        def window_body(wloc, _):
            lo = (cid * nwin_per_sc + wloc) * ws

            for r in range(_ZR):
                for cg in range(row // 16):
                    rb0[r, pl.ds(cg * 16, 16)] = zf

            def zero_body(i, _, base=sid * rpt):
                pltpu.sync_copy(rb0.at[pl.ds(0, _ZR)],
                                acc.at[pl.ds(base + i * _ZR, _ZR)])
                return 0
            lax.fori_loop(0, rpt // _ZR, zero_body, 0)
            pltpu.sync_copy(er_hbm.at[pl.ds(lo * nheads, ws * nheads)], er_win)
            plsc.subcore_barrier()

            def chunk_body(c, _, lo=lo):
                off = sid * _EPT + c * _CH
                pltpu.sync_copy(src_hbm.at[pl.ds(off, _CH)], srcv)
                pltpu.sync_copy(dst_hbm.at[pl.ds(off, _CH)], dstv)

                lane15 = jnp.full((16,), 15, jnp.int32)

                def comp_body(g, kvec):
                    d16 = dstv[pl.ds(g * 16, 16)]
                    s16 = srcv[pl.ds(g * 16, 16)]
                    m = (d16 >= lo) & (d16 < lo + ws)
                    p = jnp.where(m, 1, 0)
                    for s in (1, 2, 4, 8):
                        sh = p.at[jnp.maximum(iota - s, 0)].get(
                            mode="promise_in_bounds")
                        p = p + jnp.where(iota >= s, sh, 0)
                    trash = cap - 16 + iota
                    idx = jnp.where(m, kvec + p - 1, trash)
                    plsc.store_scatter(st_src, [idx], s16)
                    plsc.store_scatter(st_dst, [idx], d16 - lo)
                    return kvec + p.at[lane15].get(mode="promise_in_bounds")
                kvec = lax.fori_loop(0, _CH // 16, comp_body,
                                     jnp.zeros((16,), jnp.int32))
                k = kvec[0]

                for j in range(nsan):
                    sidx = k + j * 16 + iota
                    plsc.store_scatter(st_src, [sidx], zi)
                    plsc.store_scatter(st_dst, [sidx], zi)

                nb = (k + kk - 1) // kk

                for p in range(3):
                    @pl.when(p < nb)
                    def _(p=p):
                        fire(p, p)

                def tri_body(t, _, k=k):
                    for p in range(3):
                        b = 3 * t + p

                        @pl.when(b < nb)
                        def _(b=b, p=p):
                            drain(b, p)
                            process(b, p, k)

                            @pl.when(b + 3 < nb)
                            def _():
                                drain_scat(p)
                                fire(b + 3, p)
                    return 0
                lax.fori_loop(0, (nb + 2) // 3, tri_body, 0)
                for p in range(3):
                    @pl.when(p < nb)
                    def _(p=p):
                        drain_scat(p)
                return 0
            lax.fori_loop(0, _NCHUNK, chunk_body, 0)
            plsc.subcore_barrier()

            def flush_body(i, _, base=sid * rpt, lo=lo):
                o = base + i * _ZR
                pltpu.sync_copy(acc.at[pl.ds(o, _ZR)],
                                out_hbm.at[pl.ds(lo + o, _ZR)])
                return 0
            lax.fori_loop(0, rpt // _ZR, flush_body, 0)
            return 0
        lax.fori_loop(0, nwin_per_sc, window_body, 0)

    return sc_gat


_sc_gat1 = _make_sc_gat(row=144, fe=128, nheads=2, fph=64, ws=6400,
                        nwin_per_sc=4, kk=16)
_sc_gat2 = _make_sc_gat(row=48, fe=32, nheads=1, fph=32, ws=25600,
                        nwin_per_sc=1, kk=32)


def _t1_body(x_ref, w1t_ref, al_ref, ar_ref, fe_ref, er_ref):
    feat = jnp.dot(x_ref[...], w1t_ref[...], preferred_element_type=jnp.float32)
    el = jnp.dot(feat, al_ref[...], preferred_element_type=jnp.float32)
    er = jnp.dot(feat, ar_ref[...], preferred_element_type=jnp.float32)
    fe_ref[...] = jnp.concatenate(
        [feat, el, jnp.zeros((feat.shape[0], 14), jnp.float32)], axis=1)
    er_ref[...] = er


def _t2_body(acc_ref, x_ref, b1_ref, s1_ref, t1_ref, w2t_ref, al2_ref, ar2_ref,
             fe2_ref, er2_ref, h1_ref):
    a = acc_ref[...]
    d0 = jnp.maximum(a[:, 128:129], _EPS)
    d1 = jnp.maximum(a[:, 129:130], _EPS)
    rst = jnp.concatenate([a[:, 0:64] / d0, a[:, 64:128] / d1], axis=1)
    rst = rst + x_ref[...] + b1_ref[...]
    h1 = jnp.maximum(rst * s1_ref[...] + t1_ref[...], 0.0)
    feat2 = jnp.dot(h1, w2t_ref[...], preferred_element_type=jnp.float32)
    el2 = jnp.dot(feat2, al2_ref[...], preferred_element_type=jnp.float32)
    er2 = jnp.dot(feat2, ar2_ref[...], preferred_element_type=jnp.float32)
    fe2_ref[...] = jnp.concatenate(
        [feat2, el2, jnp.zeros((feat2.shape[0], 15), jnp.float32)], axis=1)
    er2_ref[...] = er2
    h1_ref[...] = h1


def kernel(x, edge_index, params):
    f32 = jnp.float32
    n = x.shape[0]
    bn_s = 1.0 / jnp.sqrt(1.0 + 1e-5)

    w1t = params['W1'].T
    al1 = params['attn_l1']
    ar1 = params['attn_r1']
    z64 = jnp.zeros((64, 1), f32)
    al1m = jnp.block([[al1[0].reshape(64, 1), z64], [z64, al1[1].reshape(64, 1)]])
    ar1m = jnp.block([[ar1[0].reshape(64, 1), z64], [z64, ar1[1].reshape(64, 1)]])
    b1 = params['bias1'].reshape(1, 128)
    s1 = (params['bn1_g'] * bn_s).reshape(1, 128)
    t1 = params['bn1_b'].reshape(1, 128)
    w2t = params['W2'].T
    al2m = params['attn_l2'].reshape(32, 1)
    ar2m = params['attn_r2'].reshape(32, 1)
    rw2t = params['resW2'].T
    b2 = params['bias2'].reshape(1, 32)
    s2 = (params['bn2_g'] * bn_s).reshape(1, 32)
    t2 = params['bn2_b'].reshape(1, 32)
    dw1t = params['dec_W1'].T
    db1 = params['dec_b1'].reshape(1, 128)
    sd = (params['dec_bn_g'] * bn_s).reshape(1, 128)
    td = params['dec_bn_b'].reshape(1, 128)
    dw2t = params['dec_W2'].T
    db2 = params['dec_b2'].reshape(1, 128)

    xp = jnp.pad(x, ((0, _NPAD - n), (0, 0)))

    grid = (_NPAD // _BLK,)
    full = lambda r, c: pl.BlockSpec((r, c), lambda i: (0, 0))
    rows = lambda c: pl.BlockSpec((_BLK, c), lambda i: (i, 0))

    fe1, er1 = pl.pallas_call(
        _t1_body,
        grid=grid,
        in_specs=[rows(128), full(128, 128), full(128, 2), full(128, 2)],
        out_specs=[rows(144), rows(2)],
        out_shape=[jax.ShapeDtypeStruct((_NPAD, 144), f32),
                   jax.ShapeDtypeStruct((_NPAD, 2), f32)],
    )(xp, w1t, al1m, ar1m)

    src = edge_index[0]
    dst = edge_index[1]
    acc1 = _sc_gat1(fe1, er1.reshape(-1), src, dst)

    fe2, er2, h1 = pl.pallas_call(
        _t2_body,
        grid=grid,
        in_specs=[rows(144), rows(128), full(1, 128), full(1, 128),
                  full(1, 128), full(128, 32), full(32, 1), full(32, 1)],
        out_specs=[rows(48), rows(1), rows(128)],
        out_shape=[jax.ShapeDtypeStruct((_NPAD, 48), f32),
                   jax.ShapeDtypeStruct((_NPAD, 1), f32),
                   jax.ShapeDtypeStruct((_NPAD, 128), f32)],
    )(acc1, xp, b1, s1, t1, w2t, al2m, ar2m)

    acc2 = _sc_gat2(fe2, er2.reshape(-1), src, dst)

    gacc, rec = pl.pallas_call(
        _t3_full_body,
        grid=grid,
        in_specs=[rows(48), rows(128), full(128, 32), full(1, 32), full(1, 32),
                  full(1, 32), full(32, 128), full(1, 128), full(1, 128),
                  full(1, 128), full(128, 128), full(1, 128)],
        out_specs=[pl.BlockSpec((8, 32), lambda i: (0, 0)), rows(128)],
        out_shape=[jax.ShapeDtypeStruct((8, 32), f32),
                   jax.ShapeDtypeStruct((_NPAD, 128), f32)],
    )(acc2, h1, rw2t, b2, s2, t2, dw1t, db1, sd, td, dw2t, db2)

    graph_rep = gacc[0:1] / float(_N)
    return (graph_rep, rec[:n])


def _t3_full_body(acc2_ref, h1_ref, rw2t_ref, b2_ref, s2_ref, t2_ref,
                  dw1t_ref, db1_ref, sd_ref, td_ref, dw2t_ref, db2_ref,
                  g_ref, rec_ref):
    i = pl.program_id(0)
    a = acc2_ref[...]
    den = jnp.maximum(a[:, 32:33], _EPS)
    rst = a[:, 0:32] / den
    rst = rst + jnp.dot(h1_ref[...], rw2t_ref[...],
                        preferred_element_type=jnp.float32) + b2_ref[...]
    h2 = jnp.maximum(rst * s2_ref[...] + t2_ref[...], 0.0)
    row = i * _BLK + lax.broadcasted_iota(jnp.int32, (_BLK, 1), 0)
    part = jnp.sum(jnp.where(row < _N, h2, 0.0), axis=0, keepdims=True)
    part8 = jnp.concatenate([part, jnp.zeros((7, 32), jnp.float32)], axis=0)

    @pl.when(i == 0)
    def _():
        g_ref[...] = jnp.zeros_like(g_ref)

    g_ref[...] += part8
    d = jnp.dot(h2, dw1t_ref[...], preferred_element_type=jnp.float32)
    d = d + db1_ref[...]
    d = jnp.maximum(d * sd_ref[...] + td_ref[...], 0.0)
    rec_ref[...] = jnp.dot(d, dw2t_ref[...],
                           preferred_element_type=jnp.float32) + db2_ref[...]
